# baseline (device time: 159939 ns/iter reference)
import jax
import jax.numpy as jnp
from jax import lax
from jax.experimental import pallas as pl
from jax.experimental.pallas import tpu as pltpu

N_DEV = 16
E_LOCAL = 4
N_EXP = N_DEV * E_LOCAL
T = 1024
D = 512
H = 1024
CAP_E = 48
SLAB = E_LOCAL * CAP_E


def _body(e_ref, x_ref, rw_ref, sw_ref, ew_ref, ltri_ref, out_ref,
          disp_ref, x_recv_ref, y_send_ref, y_recv_ref,
          x_send_sems, x_recv_sems, y_send_sems, y_recv_sems):
    me = lax.axis_index("i")

    barrier_sem = pltpu.get_barrier_semaphore()
    for p in range(N_DEV):
        @pl.when(p != me)
        def _():
            pl.semaphore_signal(barrier_sem, inc=1, device_id=(p,),
                                device_id_type=pl.DeviceIdType.MESH)
    pl.semaphore_wait(barrier_sem, N_DEV - 1)

    e = e_ref[...]
    onehot_b = (e == lax.broadcasted_iota(jnp.int32, (T, N_EXP), 1))
    onehot16 = onehot_b.astype(jnp.bfloat16)
    scores = jnp.dot(x_ref[...], rw_ref[...],
                     preferred_element_type=jnp.float32)
    smax = jnp.max(scores, axis=1, keepdims=True)
    ex = jnp.exp(scores - smax)
    probs = ex / jnp.sum(ex, axis=1, keepdims=True)
    p = jnp.sum(probs * onehot_b.astype(jnp.float32), axis=1,
                keepdims=True)
    C = jnp.dot(ltri_ref[...], onehot16,
                preferred_element_type=jnp.float32)
    rank = jnp.sum(C * onehot_b.astype(jnp.float32), axis=1,
                   keepdims=True).astype(jnp.int32)
    slot = jnp.where(rank < CAP_E, e * CAP_E + rank, -1)
    xp16 = (x_ref[...] * p).astype(jnp.bfloat16)
    colids = lax.broadcasted_iota(jnp.int32, (T, N_DEV * SLAB), 1)
    G16 = (slot == colids).astype(jnp.bfloat16)

    for o in range(N_DEV):
        m = lax.rem(me + o, N_DEV)
        ids = lax.broadcasted_iota(jnp.int32, (T, SLAB), 1) + m * SLAB
        P_t = (slot == ids).astype(jnp.bfloat16)
        disp_ref[m] = lax.dot_general(
            P_t, xp16, (((0,), (0,)), ((), ())),
            preferred_element_type=jnp.float32).astype(jnp.bfloat16)
        if o == 0:
            cp = pltpu.make_async_copy(
                disp_ref.at[m], x_recv_ref.at[m], x_recv_sems.at[m])
            cp.start()
        else:
            rdma = pltpu.make_async_remote_copy(
                src_ref=disp_ref.at[m],
                dst_ref=x_recv_ref.at[me],
                send_sem=x_send_sems.at[m],
                recv_sem=x_recv_sems.at[me],
                device_id=(m,),
                device_id_type=pl.DeviceIdType.MESH,
            )
            rdma.start()

    out_ref[...] = jnp.dot(x_ref[...].astype(jnp.bfloat16),
                           sw_ref[...].astype(jnp.bfloat16),
                           preferred_element_type=jnp.float32)

    for o in range(N_DEV):
        s = lax.rem(me - o + N_DEV, N_DEV)
        recv = pltpu.make_async_remote_copy(
            src_ref=disp_ref.at[s],
            dst_ref=x_recv_ref.at[s],
            send_sem=x_send_sems.at[s],
            recv_sem=x_recv_sems.at[s],
            device_id=(s,),
            device_id_type=pl.DeviceIdType.MESH,
        )
        recv.wait_recv()
        for j in range(E_LOCAL):
            y_send_ref[s, pl.ds(j * CAP_E, CAP_E), :] = jnp.dot(
                x_recv_ref[s, pl.ds(j * CAP_E, CAP_E), :],
                ew_ref[j],
                preferred_element_type=jnp.float32,
            ).astype(jnp.bfloat16)
        if o == 0:
            cp = pltpu.make_async_copy(
                y_send_ref.at[s], y_recv_ref.at[s], y_recv_sems.at[s])
            cp.start()
        else:
            ret = pltpu.make_async_remote_copy(
                src_ref=y_send_ref.at[s],
                dst_ref=y_recv_ref.at[me],
                send_sem=y_send_sems.at[s],
                recv_sem=y_recv_sems.at[me],
                device_id=(s,),
                device_id_type=pl.DeviceIdType.MESH,
            )
            ret.start()

    for s in range(N_DEV):
        ret = pltpu.make_async_remote_copy(
            src_ref=y_send_ref.at[s],
            dst_ref=y_recv_ref.at[s],
            send_sem=y_send_sems.at[s],
            recv_sem=y_recv_sems.at[s],
            device_id=(s,),
            device_id_type=pl.DeviceIdType.MESH,
        )
        ret.wait_recv()

    Y = y_recv_ref[...].reshape(N_DEV * SLAB, H)
    out_ref[...] += jnp.dot(G16, Y, preferred_element_type=jnp.float32)

    for o in range(1, N_DEV):
        s = lax.rem(me + o, N_DEV)
        snd = pltpu.make_async_remote_copy(
            src_ref=disp_ref.at[s],
            dst_ref=x_recv_ref.at[s],
            send_sem=x_send_sems.at[s],
            recv_sem=x_recv_sems.at[s],
            device_id=(s,),
            device_id_type=pl.DeviceIdType.MESH,
        )
        snd.wait_send()
        snd2 = pltpu.make_async_remote_copy(
            src_ref=y_send_ref.at[s],
            dst_ref=y_recv_ref.at[s],
            send_sem=y_send_sems.at[s],
            recv_sem=y_recv_sems.at[s],
            device_id=(s,),
            device_id_type=pl.DeviceIdType.MESH,
        )
        snd2.wait_send()


def kernel(x, router_W, route_idx, expert_W, shared_W):
    e = route_idx[:, :1].astype(jnp.int32)
    ew16 = expert_W.astype(jnp.bfloat16)
    ltri = jnp.tril(jnp.ones((T, T), jnp.bfloat16), -1)

    return pl.pallas_call(
        _body,
        out_shape=jax.ShapeDtypeStruct((T, H), jnp.float32),
        in_specs=[
            pl.BlockSpec(memory_space=pltpu.VMEM),
            pl.BlockSpec(memory_space=pltpu.VMEM),
            pl.BlockSpec(memory_space=pltpu.VMEM),
            pl.BlockSpec(memory_space=pltpu.VMEM),
            pl.BlockSpec(memory_space=pltpu.VMEM),
            pl.BlockSpec(memory_space=pltpu.VMEM),
        ],
        out_specs=pl.BlockSpec(memory_space=pltpu.VMEM),
        scratch_shapes=[
            pltpu.VMEM((N_DEV, SLAB, D), jnp.bfloat16),
            pltpu.VMEM((N_DEV, SLAB, D), jnp.bfloat16),
            pltpu.VMEM((N_DEV, SLAB, H), jnp.bfloat16),
            pltpu.VMEM((N_DEV, SLAB, H), jnp.bfloat16),
            pltpu.SemaphoreType.DMA((N_DEV,)),
            pltpu.SemaphoreType.DMA((N_DEV,)),
            pltpu.SemaphoreType.DMA((N_DEV,)),
            pltpu.SemaphoreType.DMA((N_DEV,)),
        ],
        compiler_params=pltpu.CompilerParams(
            collective_id=0,
            vmem_limit_bytes=100 * 1024 * 1024,
        ),
    )(e, x, router_W, shared_W, ew16, ltri)


# device time: 129192 ns/iter; 1.2380x vs baseline; 1.2380x over previous
import jax
import jax.numpy as jnp
from jax import lax
from jax.experimental import pallas as pl
from jax.experimental.pallas import tpu as pltpu

N_DEV = 16
E_LOCAL = 4
N_EXP = N_DEV * E_LOCAL
T = 1024
D = 512
H = 1024
CAP_E = 48
SLAB = E_LOCAL * CAP_E


def _body(e_ref, x_ref, rw_ref, sw_ref, ew_ref, ltri_ref, out_ref,
          disp_ref, x_recv_ref, y_send_ref, y_recv_ref,
          x_send_sems, x_recv_sems, y_send_sems, y_recv_sems):
    me = lax.axis_index("i")

    barrier_sem = pltpu.get_barrier_semaphore()
    for p in range(N_DEV):
        @pl.when(p != me)
        def _():
            pl.semaphore_signal(barrier_sem, inc=1, device_id=(p,),
                                device_id_type=pl.DeviceIdType.MESH)
    pl.semaphore_wait(barrier_sem, N_DEV - 1)

    e = e_ref[...]
    onehot_b = (e == lax.broadcasted_iota(jnp.int32, (T, N_EXP), 1))
    onehot16 = onehot_b.astype(jnp.bfloat16)
    scores = jnp.dot(x_ref[...], rw_ref[...],
                     preferred_element_type=jnp.float32)
    smax = jnp.max(scores, axis=1, keepdims=True)
    ex = jnp.exp(scores - smax)
    probs = ex / jnp.sum(ex, axis=1, keepdims=True)
    p = jnp.sum(probs * onehot_b.astype(jnp.float32), axis=1,
                keepdims=True)
    C = jnp.dot(ltri_ref[...], onehot16,
                preferred_element_type=jnp.float32)
    rank = jnp.sum(C * onehot_b.astype(jnp.float32), axis=1,
                   keepdims=True).astype(jnp.int32)
    slot = jnp.where(rank < CAP_E, e * CAP_E + rank, -1)
    xp16 = (x_ref[...] * p).astype(jnp.bfloat16)
    colids = lax.broadcasted_iota(jnp.int32, (T, N_DEV * SLAB), 1)
    G16 = (slot == colids).astype(jnp.bfloat16)

    for m in range(N_DEV):
        ids = lax.broadcasted_iota(jnp.int32, (T, SLAB), 1) + m * SLAB
        P_t = (slot == ids).astype(jnp.bfloat16)
        disp_ref[m] = lax.dot_general(
            P_t, xp16, (((0,), (0,)), ((), ())),
            preferred_element_type=jnp.float32).astype(jnp.bfloat16)

        @pl.when(m != me)
        def _():
            rdma = pltpu.make_async_remote_copy(
                src_ref=disp_ref.at[m],
                dst_ref=x_recv_ref.at[me],
                send_sem=x_send_sems.at[m],
                recv_sem=x_recv_sems.at[me],
                device_id=(m,),
                device_id_type=pl.DeviceIdType.MESH,
            )
            rdma.start()

        @pl.when(m == me)
        def _():
            cp = pltpu.make_async_copy(
                disp_ref.at[m], x_recv_ref.at[m], x_recv_sems.at[m])
            cp.start()

    out_ref[...] = jnp.dot(x_ref[...].astype(jnp.bfloat16),
                           sw_ref[...].astype(jnp.bfloat16),
                           preferred_element_type=jnp.float32)

    for s in range(N_DEV):
        recv = pltpu.make_async_remote_copy(
            src_ref=disp_ref.at[s],
            dst_ref=x_recv_ref.at[s],
            send_sem=x_send_sems.at[s],
            recv_sem=x_recv_sems.at[s],
            device_id=(s,),
            device_id_type=pl.DeviceIdType.MESH,
        )
        recv.wait_recv()
        for j in range(E_LOCAL):
            y_send_ref[s, pl.ds(j * CAP_E, CAP_E), :] = jnp.dot(
                x_recv_ref[s, pl.ds(j * CAP_E, CAP_E), :],
                ew_ref[j],
                preferred_element_type=jnp.float32,
            ).astype(jnp.bfloat16)
        @pl.when(s != me)
        def _():
            ret = pltpu.make_async_remote_copy(
                src_ref=y_send_ref.at[s],
                dst_ref=y_recv_ref.at[me],
                send_sem=y_send_sems.at[s],
                recv_sem=y_recv_sems.at[me],
                device_id=(s,),
                device_id_type=pl.DeviceIdType.MESH,
            )
            ret.start()

        @pl.when(s == me)
        def _():
            cp = pltpu.make_async_copy(
                y_send_ref.at[s], y_recv_ref.at[s], y_recv_sems.at[s])
            cp.start()

    for s in range(N_DEV):
        ret = pltpu.make_async_remote_copy(
            src_ref=y_send_ref.at[s],
            dst_ref=y_recv_ref.at[s],
            send_sem=y_send_sems.at[s],
            recv_sem=y_recv_sems.at[s],
            device_id=(s,),
            device_id_type=pl.DeviceIdType.MESH,
        )
        ret.wait_recv()

    Y = y_recv_ref[...].reshape(N_DEV * SLAB, H)
    out_ref[...] += jnp.dot(G16, Y, preferred_element_type=jnp.float32)

    for s in range(N_DEV):
        @pl.when(s != me)
        def _():
            snd = pltpu.make_async_remote_copy(
                src_ref=disp_ref.at[s],
                dst_ref=x_recv_ref.at[s],
                send_sem=x_send_sems.at[s],
                recv_sem=x_recv_sems.at[s],
                device_id=(s,),
                device_id_type=pl.DeviceIdType.MESH,
            )
            snd.wait_send()
            snd2 = pltpu.make_async_remote_copy(
                src_ref=y_send_ref.at[s],
                dst_ref=y_recv_ref.at[s],
                send_sem=y_send_sems.at[s],
                recv_sem=y_recv_sems.at[s],
                device_id=(s,),
                device_id_type=pl.DeviceIdType.MESH,
            )
            snd2.wait_send()


def kernel(x, router_W, route_idx, expert_W, shared_W):
    e = route_idx[:, :1].astype(jnp.int32)
    ew16 = expert_W.astype(jnp.bfloat16)
    ltri = jnp.tril(jnp.ones((T, T), jnp.bfloat16), -1)

    return pl.pallas_call(
        _body,
        out_shape=jax.ShapeDtypeStruct((T, H), jnp.float32),
        in_specs=[
            pl.BlockSpec(memory_space=pltpu.VMEM),
            pl.BlockSpec(memory_space=pltpu.VMEM),
            pl.BlockSpec(memory_space=pltpu.VMEM),
            pl.BlockSpec(memory_space=pltpu.VMEM),
            pl.BlockSpec(memory_space=pltpu.VMEM),
            pl.BlockSpec(memory_space=pltpu.VMEM),
        ],
        out_specs=pl.BlockSpec(memory_space=pltpu.VMEM),
        scratch_shapes=[
            pltpu.VMEM((N_DEV, SLAB, D), jnp.bfloat16),
            pltpu.VMEM((N_DEV, SLAB, D), jnp.bfloat16),
            pltpu.VMEM((N_DEV, SLAB, H), jnp.bfloat16),
            pltpu.VMEM((N_DEV, SLAB, H), jnp.bfloat16),
            pltpu.SemaphoreType.DMA((N_DEV,)),
            pltpu.SemaphoreType.DMA((N_DEV,)),
            pltpu.SemaphoreType.DMA((N_DEV,)),
            pltpu.SemaphoreType.DMA((N_DEV,)),
        ],
        compiler_params=pltpu.CompilerParams(
            collective_id=0,
            vmem_limit_bytes=100 * 1024 * 1024,
        ),
    )(e, x, router_W, shared_W, ew16, ltri)


# device time: 93482 ns/iter; 1.7109x vs baseline; 1.3820x over previous
import jax
import jax.numpy as jnp
from jax import lax
from jax.experimental import pallas as pl
from jax.experimental.pallas import tpu as pltpu

N_DEV = 16
E_LOCAL = 4
N_EXP = N_DEV * E_LOCAL
T = 1024
D = 512
H = 1024
CAP_D = 112


def _body(e_ref, x_ref, rw_ref, sw_ref, ew_ref, ltri_ref, out_ref,
          dx_ref, de_ref, x_recv_ref, e_recv_ref, y_send_ref, y_recv_ref,
          x_send_sems, x_recv_sems, e_send_sems, e_recv_sems,
          y_send_sems, y_recv_sems):
    me = lax.axis_index("i")

    barrier_sem = pltpu.get_barrier_semaphore()
    for p in range(N_DEV):
        @pl.when(p != me)
        def _():
            pl.semaphore_signal(barrier_sem, inc=1, device_id=(p,),
                                device_id_type=pl.DeviceIdType.MESH)
    pl.semaphore_wait(barrier_sem, N_DEV - 1)

    e = e_ref[...]
    dst = e // E_LOCAL
    onehot_b = (e == lax.broadcasted_iota(jnp.int32, (T, N_EXP), 1))
    oh_d_b = (dst == lax.broadcasted_iota(jnp.int32, (T, N_DEV), 1))
    oh_d16 = oh_d_b.astype(jnp.bfloat16)
    scores = jnp.dot(x_ref[...], rw_ref[...],
                     preferred_element_type=jnp.float32)
    smax = jnp.max(scores, axis=1, keepdims=True)
    ex = jnp.exp(scores - smax)
    probs = ex / jnp.sum(ex, axis=1, keepdims=True)
    p = jnp.sum(probs * onehot_b.astype(jnp.float32), axis=1,
                keepdims=True)
    C = jnp.dot(ltri_ref[...], oh_d16,
                preferred_element_type=jnp.float32)
    rank = jnp.sum(C * oh_d_b.astype(jnp.float32), axis=1,
                   keepdims=True).astype(jnp.int32)
    slot = jnp.where(rank < CAP_D, dst * CAP_D + rank, -1)
    xp16 = (x_ref[...] * p).astype(jnp.bfloat16)
    e116 = (e + 1).astype(jnp.bfloat16)
    colids = lax.broadcasted_iota(jnp.int32, (T, N_DEV * CAP_D), 1)
    G16 = (slot == colids).astype(jnp.bfloat16)

    for m in range(N_DEV):
        ids = lax.broadcasted_iota(jnp.int32, (T, CAP_D), 1) + m * CAP_D
        P_t = (slot == ids).astype(jnp.bfloat16)
        dx_ref[m] = lax.dot_general(
            P_t, xp16, (((0,), (0,)), ((), ())),
            preferred_element_type=jnp.float32).astype(jnp.bfloat16)
        de_ref[m] = lax.dot_general(
            P_t, e116, (((0,), (0,)), ((), ())),
            preferred_element_type=jnp.float32).astype(jnp.bfloat16)

        @pl.when(m != me)
        def _():
            rdma_x = pltpu.make_async_remote_copy(
                src_ref=dx_ref.at[m],
                dst_ref=x_recv_ref.at[me],
                send_sem=x_send_sems.at[m],
                recv_sem=x_recv_sems.at[me],
                device_id=(m,),
                device_id_type=pl.DeviceIdType.MESH,
            )
            rdma_x.start()
            rdma_e = pltpu.make_async_remote_copy(
                src_ref=de_ref.at[m],
                dst_ref=e_recv_ref.at[me],
                send_sem=e_send_sems.at[m],
                recv_sem=e_recv_sems.at[me],
                device_id=(m,),
                device_id_type=pl.DeviceIdType.MESH,
            )
            rdma_e.start()

        @pl.when(m == me)
        def _():
            cpx = pltpu.make_async_copy(
                dx_ref.at[m], x_recv_ref.at[m], x_recv_sems.at[m])
            cpx.start()
            cpe = pltpu.make_async_copy(
                de_ref.at[m], e_recv_ref.at[m], e_recv_sems.at[m])
            cpe.start()

    out_ref[...] = jnp.dot(x_ref[...].astype(jnp.bfloat16),
                           sw_ref[...].astype(jnp.bfloat16),
                           preferred_element_type=jnp.float32)

    for s in range(N_DEV):
        recv_x = pltpu.make_async_remote_copy(
            src_ref=dx_ref.at[s],
            dst_ref=x_recv_ref.at[s],
            send_sem=x_send_sems.at[s],
            recv_sem=x_recv_sems.at[s],
            device_id=(s,),
            device_id_type=pl.DeviceIdType.MESH,
        )
        recv_x.wait_recv()
        recv_e = pltpu.make_async_remote_copy(
            src_ref=de_ref.at[s],
            dst_ref=e_recv_ref.at[s],
            send_sem=e_send_sems.at[s],
            recv_sem=e_recv_sems.at[s],
            device_id=(s,),
            device_id_type=pl.DeviceIdType.MESH,
        )
        recv_e.wait_recv()

        eid = e_recv_ref[s].astype(jnp.int32)
        y = jnp.zeros((CAP_D, H), jnp.float32)
        for j in range(E_LOCAL):
            yj = jnp.dot(x_recv_ref[s], ew_ref[j],
                         preferred_element_type=jnp.float32)
            mask = eid == (me * E_LOCAL + j + 1)
            y = y + jnp.where(mask, yj, 0.0)
        y_send_ref[s] = y.astype(jnp.bfloat16)

        @pl.when(s != me)
        def _():
            ret = pltpu.make_async_remote_copy(
                src_ref=y_send_ref.at[s],
                dst_ref=y_recv_ref.at[me],
                send_sem=y_send_sems.at[s],
                recv_sem=y_recv_sems.at[me],
                device_id=(s,),
                device_id_type=pl.DeviceIdType.MESH,
            )
            ret.start()

        @pl.when(s == me)
        def _():
            cp = pltpu.make_async_copy(
                y_send_ref.at[s], y_recv_ref.at[s], y_recv_sems.at[s])
            cp.start()

    for s in range(N_DEV):
        ret = pltpu.make_async_remote_copy(
            src_ref=y_send_ref.at[s],
            dst_ref=y_recv_ref.at[s],
            send_sem=y_send_sems.at[s],
            recv_sem=y_recv_sems.at[s],
            device_id=(s,),
            device_id_type=pl.DeviceIdType.MESH,
        )
        ret.wait_recv()

    Y = y_recv_ref[...].reshape(N_DEV * CAP_D, H)
    out_ref[...] += jnp.dot(G16, Y, preferred_element_type=jnp.float32)

    for s in range(N_DEV):
        @pl.when(s != me)
        def _():
            for src, dst_r, sems in (
                (dx_ref, x_recv_ref, x_send_sems),
                (de_ref, e_recv_ref, e_send_sems),
                (y_send_ref, y_recv_ref, y_send_sems),
            ):
                snd = pltpu.make_async_remote_copy(
                    src_ref=src.at[s],
                    dst_ref=dst_r.at[s],
                    send_sem=sems.at[s],
                    recv_sem=x_recv_sems.at[s],
                    device_id=(s,),
                    device_id_type=pl.DeviceIdType.MESH,
                )
                snd.wait_send()


def kernel(x, router_W, route_idx, expert_W, shared_W):
    e = route_idx[:, :1].astype(jnp.int32)
    ew16 = expert_W.astype(jnp.bfloat16)
    ltri = jnp.tril(jnp.ones((T, T), jnp.bfloat16), -1)

    return pl.pallas_call(
        _body,
        out_shape=jax.ShapeDtypeStruct((T, H), jnp.float32),
        in_specs=[
            pl.BlockSpec(memory_space=pltpu.VMEM),
            pl.BlockSpec(memory_space=pltpu.VMEM),
            pl.BlockSpec(memory_space=pltpu.VMEM),
            pl.BlockSpec(memory_space=pltpu.VMEM),
            pl.BlockSpec(memory_space=pltpu.VMEM),
            pl.BlockSpec(memory_space=pltpu.VMEM),
        ],
        out_specs=pl.BlockSpec(memory_space=pltpu.VMEM),
        scratch_shapes=[
            pltpu.VMEM((N_DEV, CAP_D, D), jnp.bfloat16),
            pltpu.VMEM((N_DEV, CAP_D, 1), jnp.bfloat16),
            pltpu.VMEM((N_DEV, CAP_D, D), jnp.bfloat16),
            pltpu.VMEM((N_DEV, CAP_D, 1), jnp.bfloat16),
            pltpu.VMEM((N_DEV, CAP_D, H), jnp.bfloat16),
            pltpu.VMEM((N_DEV, CAP_D, H), jnp.bfloat16),
            pltpu.SemaphoreType.DMA((N_DEV,)),
            pltpu.SemaphoreType.DMA((N_DEV,)),
            pltpu.SemaphoreType.DMA((N_DEV,)),
            pltpu.SemaphoreType.DMA((N_DEV,)),
            pltpu.SemaphoreType.DMA((N_DEV,)),
            pltpu.SemaphoreType.DMA((N_DEV,)),
        ],
        compiler_params=pltpu.CompilerParams(
            collective_id=0,
            vmem_limit_bytes=100 * 1024 * 1024,
        ),
    )(e, x, router_W, shared_W, ew16, ltri)


# device time: 91782 ns/iter; 1.7426x vs baseline; 1.0185x over previous
import jax
import jax.numpy as jnp
from jax import lax
from jax.experimental import pallas as pl
from jax.experimental.pallas import tpu as pltpu

N_DEV = 16
E_LOCAL = 4
N_EXP = N_DEV * E_LOCAL
T = 1024
D = 512
H = 1024
CAP_D = 112


def _body(e_ref, x_ref, rw_ref, sw_ref, ew_ref, ltri_ref, out_ref,
          dx_ref, de_ref, x_recv_ref, e_recv_ref, y_send_ref, y_recv_ref,
          x_send_sems, x_recv_sems, e_send_sems, e_recv_sems,
          y_send_sems, y_recv_sems):
    me = lax.axis_index("i")

    barrier_sem = pltpu.get_barrier_semaphore()
    for p in range(N_DEV):
        @pl.when(p != me)
        def _():
            pl.semaphore_signal(barrier_sem, inc=1, device_id=(p,),
                                device_id_type=pl.DeviceIdType.MESH)
    pl.semaphore_wait(barrier_sem, N_DEV - 1)

    e = e_ref[...]
    dst = e // E_LOCAL
    onehot_b = (e == lax.broadcasted_iota(jnp.int32, (T, N_EXP), 1))
    oh_d_b = (dst == lax.broadcasted_iota(jnp.int32, (T, N_DEV), 1))
    oh_d16 = oh_d_b.astype(jnp.bfloat16)
    scores = jnp.dot(x_ref[...], rw_ref[...],
                     preferred_element_type=jnp.float32)
    smax = jnp.max(scores, axis=1, keepdims=True)
    ex = jnp.exp(scores - smax)
    probs = ex / jnp.sum(ex, axis=1, keepdims=True)
    p = jnp.sum(probs * onehot_b.astype(jnp.float32), axis=1,
                keepdims=True)
    C = jnp.dot(ltri_ref[...], oh_d16,
                preferred_element_type=jnp.float32)
    rank = jnp.sum(C * oh_d_b.astype(jnp.float32), axis=1,
                   keepdims=True).astype(jnp.int32)
    slot = jnp.where(rank < CAP_D, dst * CAP_D + rank, -1)
    xp16 = (x_ref[...] * p).astype(jnp.bfloat16)
    e116 = (e + 1).astype(jnp.bfloat16)

    for m in range(N_DEV):
        ids = lax.broadcasted_iota(jnp.int32, (T, CAP_D), 1) + m * CAP_D
        P_t = (slot == ids).astype(jnp.bfloat16)
        dx_ref[m] = lax.dot_general(
            P_t, xp16, (((0,), (0,)), ((), ())),
            preferred_element_type=jnp.float32).astype(jnp.bfloat16)
        de_ref[m] = lax.dot_general(
            P_t, e116, (((0,), (0,)), ((), ())),
            preferred_element_type=jnp.float32).astype(jnp.bfloat16)

        @pl.when(m != me)
        def _():
            rdma_x = pltpu.make_async_remote_copy(
                src_ref=dx_ref.at[m],
                dst_ref=x_recv_ref.at[me],
                send_sem=x_send_sems.at[m],
                recv_sem=x_recv_sems.at[me],
                device_id=(m,),
                device_id_type=pl.DeviceIdType.MESH,
            )
            rdma_x.start()
            rdma_e = pltpu.make_async_remote_copy(
                src_ref=de_ref.at[m],
                dst_ref=e_recv_ref.at[me],
                send_sem=e_send_sems.at[m],
                recv_sem=e_recv_sems.at[me],
                device_id=(m,),
                device_id_type=pl.DeviceIdType.MESH,
            )
            rdma_e.start()

        @pl.when(m == me)
        def _():
            cpx = pltpu.make_async_copy(
                dx_ref.at[m], x_recv_ref.at[m], x_recv_sems.at[m])
            cpx.start()
            cpe = pltpu.make_async_copy(
                de_ref.at[m], e_recv_ref.at[m], e_recv_sems.at[m])
            cpe.start()

    colids = lax.broadcasted_iota(jnp.int32, (T, N_DEV * CAP_D), 1)
    G16 = (slot == colids).astype(jnp.bfloat16)
    out_ref[...] = jnp.dot(x_ref[...].astype(jnp.bfloat16),
                           sw_ref[...].astype(jnp.bfloat16),
                           preferred_element_type=jnp.float32)

    for s in range(N_DEV):
        recv_x = pltpu.make_async_remote_copy(
            src_ref=dx_ref.at[s],
            dst_ref=x_recv_ref.at[s],
            send_sem=x_send_sems.at[s],
            recv_sem=x_recv_sems.at[s],
            device_id=(s,),
            device_id_type=pl.DeviceIdType.MESH,
        )
        recv_x.wait_recv()
        recv_e = pltpu.make_async_remote_copy(
            src_ref=de_ref.at[s],
            dst_ref=e_recv_ref.at[s],
            send_sem=e_send_sems.at[s],
            recv_sem=e_recv_sems.at[s],
            device_id=(s,),
            device_id_type=pl.DeviceIdType.MESH,
        )
        recv_e.wait_recv()

        eid = e_recv_ref[s].astype(jnp.int32)
        y = jnp.zeros((CAP_D, H), jnp.float32)
        for j in range(E_LOCAL):
            yj = jnp.dot(x_recv_ref[s], ew_ref[j],
                         preferred_element_type=jnp.float32)
            mask = eid == (me * E_LOCAL + j + 1)
            y = y + jnp.where(mask, yj, 0.0)
        y_send_ref[s] = y.astype(jnp.bfloat16)

        @pl.when(s != me)
        def _():
            ret = pltpu.make_async_remote_copy(
                src_ref=y_send_ref.at[s],
                dst_ref=y_recv_ref.at[me],
                send_sem=y_send_sems.at[s],
                recv_sem=y_recv_sems.at[me],
                device_id=(s,),
                device_id_type=pl.DeviceIdType.MESH,
            )
            ret.start()

        @pl.when(s == me)
        def _():
            cp = pltpu.make_async_copy(
                y_send_ref.at[s], y_recv_ref.at[s], y_recv_sems.at[s])
            cp.start()

    HALF = N_DEV // 2
    for half in range(2):
        for s in range(half * HALF, (half + 1) * HALF):
            ret = pltpu.make_async_remote_copy(
                src_ref=y_send_ref.at[s],
                dst_ref=y_recv_ref.at[s],
                send_sem=y_send_sems.at[s],
                recv_sem=y_recv_sems.at[s],
                device_id=(s,),
                device_id_type=pl.DeviceIdType.MESH,
            )
            ret.wait_recv()
        Yh = y_recv_ref[half * HALF:(half + 1) * HALF].reshape(
            HALF * CAP_D, H)
        Gh = G16[:, half * HALF * CAP_D:(half + 1) * HALF * CAP_D]
        out_ref[...] += jnp.dot(Gh, Yh, preferred_element_type=jnp.float32)

    for s in range(N_DEV):
        @pl.when(s != me)
        def _():
            for src, dst_r, sems in (
                (dx_ref, x_recv_ref, x_send_sems),
                (de_ref, e_recv_ref, e_send_sems),
                (y_send_ref, y_recv_ref, y_send_sems),
            ):
                snd = pltpu.make_async_remote_copy(
                    src_ref=src.at[s],
                    dst_ref=dst_r.at[s],
                    send_sem=sems.at[s],
                    recv_sem=x_recv_sems.at[s],
                    device_id=(s,),
                    device_id_type=pl.DeviceIdType.MESH,
                )
                snd.wait_send()


def kernel(x, router_W, route_idx, expert_W, shared_W):
    e = route_idx[:, :1].astype(jnp.int32)
    ew16 = expert_W.astype(jnp.bfloat16)
    ltri = jnp.tril(jnp.ones((T, T), jnp.bfloat16), -1)

    return pl.pallas_call(
        _body,
        out_shape=jax.ShapeDtypeStruct((T, H), jnp.float32),
        in_specs=[
            pl.BlockSpec(memory_space=pltpu.VMEM),
            pl.BlockSpec(memory_space=pltpu.VMEM),
            pl.BlockSpec(memory_space=pltpu.VMEM),
            pl.BlockSpec(memory_space=pltpu.VMEM),
            pl.BlockSpec(memory_space=pltpu.VMEM),
            pl.BlockSpec(memory_space=pltpu.VMEM),
        ],
        out_specs=pl.BlockSpec(memory_space=pltpu.VMEM),
        scratch_shapes=[
            pltpu.VMEM((N_DEV, CAP_D, D), jnp.bfloat16),
            pltpu.VMEM((N_DEV, CAP_D, 1), jnp.bfloat16),
            pltpu.VMEM((N_DEV, CAP_D, D), jnp.bfloat16),
            pltpu.VMEM((N_DEV, CAP_D, 1), jnp.bfloat16),
            pltpu.VMEM((N_DEV, CAP_D, H), jnp.bfloat16),
            pltpu.VMEM((N_DEV, CAP_D, H), jnp.bfloat16),
            pltpu.SemaphoreType.DMA((N_DEV,)),
            pltpu.SemaphoreType.DMA((N_DEV,)),
            pltpu.SemaphoreType.DMA((N_DEV,)),
            pltpu.SemaphoreType.DMA((N_DEV,)),
            pltpu.SemaphoreType.DMA((N_DEV,)),
            pltpu.SemaphoreType.DMA((N_DEV,)),
        ],
        compiler_params=pltpu.CompilerParams(
            collective_id=0,
            vmem_limit_bytes=100 * 1024 * 1024,
        ),
    )(e, x, router_W, shared_W, ew16, ltri)
